# Initial kernel scaffold; baseline (speedup 1.0000x reference)
import jax
import jax.numpy as jnp
from jax.experimental import pallas as pl


def kernel(x, edge_index, W1, al1, ar1, b1, W2, al2, ar2, b2):
    raise NotImplementedError("write your pallas kernel here")


# speedup vs baseline: 1.0157x; 1.0157x over previous
import functools

import jax
import jax.numpy as jnp
from jax.experimental import pallas as pl
from jax.experimental.pallas import tpu as pltpu


def _matmul_kernel(x_ref, w_ref, o_ref):
    o_ref[...] = jnp.dot(x_ref[...], w_ref[...],
                         preferred_element_type=jnp.float32)


def _pallas_matmul(x, w, bm=1000):
    m, k = x.shape
    k2, n = w.shape
    assert k == k2 and m % bm == 0
    return pl.pallas_call(
        _matmul_kernel,
        grid=(m // bm,),
        in_specs=[
            pl.BlockSpec((bm, k), lambda i: (i, 0)),
            pl.BlockSpec((k, n), lambda i: (0, 0)),
        ],
        out_specs=pl.BlockSpec((bm, n), lambda i: (i, 0)),
        out_shape=jax.ShapeDtypeStruct((m, n), jnp.float32),
    )(x, w)


def _edge_softmax_agg(table, src, dst, el, er, n):
    e = jax.nn.leaky_relu(el[src] + er[dst], negative_slope=0.2)
    emax = jax.ops.segment_max(e, dst, num_segments=n)
    emax = jnp.where(jnp.isfinite(emax), emax, 0.0)
    ee = jnp.exp(e - emax[dst])
    denom = jax.ops.segment_sum(ee, dst, num_segments=n)
    alpha = ee / (denom[dst] + 1e-9)
    return alpha


def kernel(x, edge_index, W1, al1, ar1, b1, W2, al2, ar2, b2):
    n, d = x.shape
    h = al1.shape[1]
    dh1 = al1.shape[2]
    d2 = al2.shape[2]
    src = edge_index[0]
    dst = edge_index[1]

    feat1 = _pallas_matmul(x, W1)
    f1 = feat1.reshape(n, h, dh1)
    el1 = jnp.einsum("nhd,hd->nh", f1, al1[0])
    er1 = jnp.einsum("nhd,hd->nh", f1, ar1[0])
    alpha1 = _edge_softmax_agg(feat1, src, dst, el1, er1, n)
    msg1 = f1[src] * alpha1[:, :, None]
    agg1 = jax.ops.segment_sum(msg1, dst, num_segments=n)
    out1 = (agg1 + b1.reshape(1, h, dh1)).reshape(n, h * dh1)
    h1 = x + jax.nn.elu(out1)

    w2h = W2.reshape(d, h, d2)
    vl2 = jnp.einsum("dhk,hk->dh", w2h, al2[0])
    vr2 = jnp.einsum("dhk,hk->dh", w2h, ar2[0])
    el2 = h1 @ vl2
    er2 = h1 @ vr2
    alpha2 = _edge_softmax_agg(h1, src, dst, el2, er2, n)
    msg2 = h1[src][:, None, :] * alpha2[:, :, None]
    agg2 = jax.ops.segment_sum(msg2, dst, num_segments=n)
    w2r = w2h.transpose(1, 0, 2).reshape(h * d, d2)
    h2 = _pallas_matmul(agg2.reshape(n, h * d), w2r) / h
    h2 = h2 + b2.reshape(h, d2).mean(axis=0)[None, :]
    return h1 + h2
